# baseline (speedup 1.0000x reference)
import jax
import jax.numpy as jnp
from jax import lax
from jax.experimental import pallas as pl
from jax.experimental.pallas import tpu as pltpu
from jax.experimental.pallas import tpu_sc as plsc
from jax._src.pallas import mpmd

_B, _C, _H, _W = 8, 192, 224, 224
_ROWS = _B * _C
_D = _H * _W
_NC, _NS = 2, 16

_SROWS = 768
_SRPC = _SROWS // _NC
_G = 4
_SNB = 2
_SNGRP = _SRPC // _G

_TROWS = _ROWS - _SROWS
_NW = _NC * _NS
_TRPW = _TROWS // _NW
_SPLIT = 2
_CH = _D // _SPLIT
_TNB = 4
_TT = _TRPW * _SPLIT

_smesh = plsc.ScalarSubcoreMesh(axis_name="c", num_cores=_NC)
_vmesh = plsc.VectorSubcoreMesh(core_axis_name="c", subcore_axis_name="s")


def _src_of(r):
    b = r // _C
    c = lax.rem(r, _C)
    return b * _C + (_C - 1 - c)


def _scs_fn(in_hbm, out_hbm, sbufs, s_in, s_out, tbufs, t_in, t_out):
    del tbufs, t_in, t_out
    cid = lax.axis_index("c")
    row0 = cid * _SRPC

    def grp_info(g):
        r0 = row0 + g * _G
        b = r0 // _C
        c0 = lax.rem(r0, _C)
        s0 = b * _C + (_C - 1 - c0 - (_G - 1))
        return r0, s0

    def start_in(g):
        slot = lax.rem(g, _SNB)
        _, s0 = grp_info(g)
        pltpu.async_copy(in_hbm.at[pl.ds(s0, _G)], sbufs.at[slot],
                         s_in.at[slot])

    def wait_in(g):
        slot = lax.rem(g, _SNB)
        _, s0 = grp_info(g)
        pltpu.make_async_copy(in_hbm.at[pl.ds(s0, _G)], sbufs.at[slot],
                              s_in.at[slot]).wait()

    def start_outs(g):
        slot = lax.rem(g, _SNB)
        r0, _ = grp_info(g)
        for j in range(_G):
            pltpu.async_copy(sbufs.at[slot, _G - 1 - j], out_hbm.at[r0 + j],
                             s_out.at[slot])

    def wait_outs(g):
        slot = lax.rem(g, _SNB)
        r0, _ = grp_info(g)
        for j in range(_G):
            pltpu.make_async_copy(sbufs.at[slot, _G - 1 - j],
                                  out_hbm.at[r0 + j], s_out.at[slot]).wait()

    for j in range(_SNB - 1):
        start_in(j)

    def body(g, carry):
        wait_in(g)
        start_outs(g)
        pl.when(jnp.logical_and(g >= 1, g + _SNB - 1 < _SNGRP))(
            lambda: wait_outs(g - 1))
        pl.when(g + _SNB - 1 < _SNGRP)(lambda: start_in(g + _SNB - 1))
        return carry

    lax.fori_loop(0, _SNGRP, body, 0)
    for j in range(_SNGRP - _SNB, _SNGRP):
        wait_outs(j)


def _tec_fn(in_hbm, out_hbm, sbufs, s_in, s_out, tbufs, t_in, t_out):
    del sbufs, s_in, s_out
    cid = lax.axis_index("c")
    sid = lax.axis_index("s")
    wid = sid * _NC + cid
    base = _SROWS + wid * _TRPW

    def src_slice(i):
        r = base + i // _SPLIT
        k = lax.rem(i, _SPLIT)
        return in_hbm.at[_src_of(r), pl.ds(k * _CH, _CH)]

    def dst_slice(i):
        r = base + i // _SPLIT
        k = lax.rem(i, _SPLIT)
        return out_hbm.at[r, pl.ds(k * _CH, _CH)]

    def start_in(i):
        slot = lax.rem(i, _TNB)
        pltpu.async_copy(src_slice(i), tbufs.at[slot], t_in.at[slot])

    def wait_in(i):
        slot = lax.rem(i, _TNB)
        pltpu.make_async_copy(src_slice(i), tbufs.at[slot],
                              t_in.at[slot]).wait()

    def start_out(i):
        slot = lax.rem(i, _TNB)
        pltpu.async_copy(tbufs.at[slot], dst_slice(i), t_out.at[slot])

    def wait_out(i):
        slot = lax.rem(i, _TNB)
        pltpu.make_async_copy(tbufs.at[slot], dst_slice(i),
                              t_out.at[slot]).wait()

    for j in range(_TNB - 1):
        start_in(j)

    def body(i, carry):
        wait_in(i)
        start_out(i)
        pl.when(jnp.logical_and(i >= 1, i + _TNB - 1 < _TT))(
            lambda: wait_out(i - 1))
        pl.when(i + _TNB - 1 < _TT)(lambda: start_in(i + _TNB - 1))
        return carry

    lax.fori_loop(0, _TT, body, 0)
    for j in range(_TT - _TNB, _TT):
        wait_out(j)


_SEM = pltpu.MemorySpace.SEMAPHORE
_reverse_rows = mpmd.mpmd_map(
    [(_smesh, _scs_fn), (_vmesh, _tec_fn)],
    out_types=[jax.ShapeDtypeStruct((_ROWS, _D), jnp.float32)],
    scratch_types=[
        pltpu.VMEM_SHARED((_SNB, _G, _D), jnp.float32),
        (_SEM @ _smesh)((_SNB,), pltpu.SemaphoreType.DMA.dtype),
        (_SEM @ _smesh)((_SNB,), pltpu.SemaphoreType.DMA.dtype),
        (pltpu.VMEM @ _vmesh)((_TNB, _CH), jnp.float32),
        (_SEM @ _vmesh)((_TNB,), pltpu.SemaphoreType.DMA.dtype),
        (_SEM @ _vmesh)((_TNB,), pltpu.SemaphoreType.DMA.dtype),
    ],
)


def kernel(input):
    x = input.reshape(_ROWS, _D)
    (y,) = _reverse_rows(x)
    return y.reshape(_B, _C, _H, _W)
